# baseline (speedup 1.0000x reference)
import jax
import jax.numpy as jnp
from jax import lax
from jax.experimental import pallas as pl
from jax.experimental.pallas import tpu as pltpu
from jax.experimental.pallas import tpu_sc as plsc

D_EMB = 64
NC, NS = 2, 16
NW = NC * NS
NB = 2
SPLITS = ((0, 128), (128, 72))

VOCAB = 1000000
LANES = 16
BLK = 128
CHUNK = 4
FULL_BLOCKS = VOCAB // BLK
TAIL = VOCAB - FULL_BLOCKS * BLK
N_CHUNKS = FULL_BLOCKS // CHUNK


N_TILES = CHUNK * 8


def _transpose_body(tabT_hbm, out_hbm, in_v, out_v, in64_v, out64_v, isem):
    wid = lax.axis_index("s") * NC + lax.axis_index("c")

    iota = lax.iota(jnp.int32, LANES)
    s_pat = iota & 7
    l_pats = [(iota + k) & (LANES - 1) for k in range(8)]
    dst_pats = [l_pats[k] * D_EMB + s_pat for k in range(8)]

    def transpose_tiles(src_v, dstf_v, n_it, l_chunks):
        def sub(it, carry):
            t = it // l_chunks
            l0 = it % l_chunks
            tr = t // CHUNK
            tc = t % CHUNK
            t_bc = s_pat * 0 + t
            obase = tc * (BLK * D_EMB) + l0 * (LANES * D_EMB) + tr * 8
            lbase = l0 * LANES
            for k in range(8):
                v = plsc.load_gather(src_v, [t_bc, s_pat,
                                             lbase + l_pats[k]])
                plsc.store_scatter(dstf_v, [obase + dst_pats[k]], v)
            return carry
        lax.fori_loop(0, n_it, sub, 0)

    def chunk_body(jj, carry):
        ch = wid + jj * NW

        def do_chunk():
            col0 = ch * (CHUNK * BLK)
            descs = []
            for tr in range(8):
                for tc in range(CHUNK):
                    descs.append(pltpu.async_copy(
                        tabT_hbm.at[pl.ds(tr * 8, 8),
                                    pl.ds(col0 + tc * BLK, BLK)],
                        in_v.at[tr * CHUNK + tc], isem))
            for d in descs:
                d.wait()
            transpose_tiles(in_v, out_v, N_TILES * 8, 8)
            pltpu.sync_copy(out_v,
                            out_hbm.at[pl.ds(col0 * D_EMB,
                                             CHUNK * BLK * D_EMB)])
        pl.when(ch < N_CHUNKS)(do_chunk)
        return carry

    lax.fori_loop(0, (N_CHUNKS + NW - 1) // NW, chunk_body, 0)

    def do_tail():
        col0 = FULL_BLOCKS * BLK
        descs = []
        for tr in range(8):
            descs.append(pltpu.async_copy(
                tabT_hbm.at[pl.ds(tr * 8, 8), pl.ds(col0, TAIL)],
                in64_v.at[tr], isem))
        for d in descs:
            d.wait()

        def sub(it, carry):
            tr = it // 4
            l0 = it % 4
            t_bc = s_pat * 0 + tr
            obase = l0 * (LANES * D_EMB) + tr * 8
            lbase = l0 * LANES
            for k in range(8):
                v = plsc.load_gather(in64_v, [t_bc, s_pat,
                                              lbase + l_pats[k]])
                plsc.store_scatter(out64_v, [obase + dst_pats[k]], v)
            return carry
        lax.fori_loop(0, 32, sub, 0)
        pltpu.sync_copy(out64_v,
                        out_hbm.at[pl.ds(col0 * D_EMB, TAIL * D_EMB)])
    pl.when(wid == NW - 1)(do_tail)


def _gather_body(idx_hbm, table_hbm, out_hbm, idx_v, rows_v,
                 gsem0, gsem1, wsem0, wsem1):
    wid = lax.axis_index("s") * NC + lax.axis_index("c")
    batch_per_w = idx_v.shape[0]
    seq = idx_v.shape[1]
    grows = NB * seq
    groups = batch_per_w // NB
    pairs = groups // 2
    b_base = wid * batch_per_w
    row_base = b_base * seq

    pltpu.sync_copy(idx_hbm.at[pl.ds(b_base, batch_per_w)], idx_v)

    def issue_gathers(g, buf, sem):
        for r in range(NB):
            for off, ln in SPLITS:
                pltpu.async_copy(
                    table_hbm.at[idx_v.at[g * NB + r, pl.ds(off, ln)]],
                    rows_v.at[buf, pl.ds(r * seq + off, ln)],
                    sem)

    def drain(sem, buf):
        pltpu.make_async_copy(out_hbm.at[pl.ds(0, grows), pl.ds(0, D_EMB)],
                              rows_v.at[buf], sem).wait()

    def writeback(g, buf, sem):
        pltpu.async_copy(rows_v.at[buf],
                         out_hbm.at[pl.ds(row_base + g * grows, grows),
                                    pl.ds(0, D_EMB)], sem)

    issue_gathers(0, 0, gsem0)

    def body(i, carry):
        a = 2 * i
        drain(gsem0, 0)
        writeback(a, 0, wsem0)
        pl.when(i > 0)(lambda: drain(wsem1, 1))
        issue_gathers(a + 1, 1, gsem1)

        def advance():
            drain(wsem0, 0)
            issue_gathers(a + 2, 0, gsem0)
        pl.when(i < pairs - 1)(advance)

        drain(gsem1, 1)
        writeback(a + 1, 1, wsem1)
        return carry

    lax.fori_loop(0, pairs, body, 0)
    drain(wsem0, 0)
    drain(wsem1, 1)


@jax.jit
def kernel(input_ids, table):
    batch, seq = input_ids.shape
    assert batch % NW == 0 and (batch // NW) % (2 * NB) == 0
    batch_per_w = batch // NW

    mesh = plsc.VectorSubcoreMesh(core_axis_name="c", subcore_axis_name="s",
                                  num_cores=NC, num_subcores=NS)

    run_t = pl.kernel(
        _transpose_body,
        out_type=jax.ShapeDtypeStruct((VOCAB * D_EMB,), jnp.float32),
        mesh=mesh,
        scratch_types=[
            pltpu.VMEM((N_TILES, 8, BLK), jnp.float32),
            pltpu.VMEM((CHUNK * BLK * D_EMB,), jnp.float32),
            pltpu.VMEM((8, 8, TAIL), jnp.float32),
            pltpu.VMEM((TAIL * D_EMB,), jnp.float32),
            pltpu.SemaphoreType.DMA,
        ],
        compiler_params=pltpu.CompilerParams(use_tc_tiling_on_sc=True,
                                             needs_layout_passes=False),
    )
    table_lin = run_t(table.T).reshape(VOCAB, D_EMB)

    run = pl.kernel(
        _gather_body,
        out_type=jax.ShapeDtypeStruct((batch * seq, 2 * D_EMB), jnp.float32),
        mesh=mesh,
        scratch_types=[
            pltpu.VMEM((batch_per_w, seq), jnp.int32),
            pltpu.VMEM((2, NB * seq, D_EMB), jnp.float32),
            pltpu.SemaphoreType.DMA,
            pltpu.SemaphoreType.DMA,
            pltpu.SemaphoreType.DMA,
            pltpu.SemaphoreType.DMA,
        ],
        compiler_params=pltpu.CompilerParams(use_tc_tiling_on_sc=False),
    )
    out = run(input_ids.astype(jnp.int32), table_lin)
    return out[:, :D_EMB].reshape(batch, seq, D_EMB)


# speedup vs baseline: 1.0001x; 1.0001x over previous
import jax
import jax.numpy as jnp
from jax import lax
from jax.experimental import pallas as pl
from jax.experimental.pallas import tpu as pltpu
from jax.experimental.pallas import tpu_sc as plsc

D_EMB = 64
NC, NS = 2, 16
NW = NC * NS
NB = 2
SPLITS = ((0, 128), (128, 72))

VOCAB = 1000000
LANES = 16
BLK = 128
CHUNK = 4
FULL_BLOCKS = VOCAB // BLK
TAIL = VOCAB - FULL_BLOCKS * BLK
N_CHUNKS = FULL_BLOCKS // CHUNK


N_TILES = CHUNK * 8


def _transpose_body(tabT_hbm, out_hbm, in_v, out_v, in64_v, out64_v, isem):
    wid = lax.axis_index("s") * NC + lax.axis_index("c")

    iota = lax.iota(jnp.int32, LANES)
    s_pat = iota & 7
    l_pats = [(iota + k) & (LANES - 1) for k in range(8)]
    dst_pats = [l_pats[k] * D_EMB + s_pat for k in range(8)]

    def transpose_tiles(src_v, dstf_v, n_it, l_chunks):
        def sub(it, carry):
            t = it // l_chunks
            l0 = it % l_chunks
            tr = t // CHUNK
            tc = t % CHUNK
            t_bc = s_pat * 0 + t
            obase = tc * (BLK * D_EMB) + l0 * (LANES * D_EMB) + tr * 8
            lbase = l0 * LANES
            for k in range(8):
                v = plsc.load_gather(src_v, [t_bc, s_pat,
                                             lbase + l_pats[k]])
                plsc.store_scatter(dstf_v, [obase + dst_pats[k]], v)
            return carry
        lax.fori_loop(0, n_it, sub, 0)

    def chunk_body(jj, carry):
        ch = wid + jj * NW

        def do_chunk():
            col0 = ch * (CHUNK * BLK)
            descs = []
            for tr in range(8):
                for tc in range(CHUNK):
                    descs.append(pltpu.async_copy(
                        tabT_hbm.at[pl.ds(tr * 8, 8),
                                    pl.ds(col0 + tc * BLK, BLK)],
                        in_v.at[tr * CHUNK + tc], isem))
            for d in descs:
                d.wait()
            transpose_tiles(in_v, out_v, N_TILES * 8, 8)
            pltpu.sync_copy(out_v,
                            out_hbm.at[pl.ds(col0 * D_EMB,
                                             CHUNK * BLK * D_EMB)])
        pl.when(ch < N_CHUNKS)(do_chunk)
        return carry

    lax.fori_loop(0, (N_CHUNKS + NW - 1) // NW, chunk_body, 0)

    def do_tail():
        col0 = FULL_BLOCKS * BLK
        descs = []
        for tr in range(8):
            descs.append(pltpu.async_copy(
                tabT_hbm.at[pl.ds(tr * 8, 8), pl.ds(col0, TAIL)],
                in64_v.at[tr], isem))
        for d in descs:
            d.wait()

        def sub(it, carry):
            tr = it // 4
            l0 = it % 4
            t_bc = s_pat * 0 + tr
            obase = l0 * (LANES * D_EMB) + tr * 8
            lbase = l0 * LANES
            for k in range(8):
                v = plsc.load_gather(in64_v, [t_bc, s_pat,
                                              lbase + l_pats[k]])
                plsc.store_scatter(out64_v, [obase + dst_pats[k]], v)
            return carry
        lax.fori_loop(0, 32, sub, 0)
        pltpu.sync_copy(out64_v,
                        out_hbm.at[pl.ds(col0 * D_EMB, TAIL * D_EMB)])
    pl.when(wid == NW - 1)(do_tail)


def _gather_body(idx_hbm, table_hbm, out_hbm, idx_v, rows_v,
                 gsem0, gsem1, wsem0, wsem1):
    wid = lax.axis_index("s") * NC + lax.axis_index("c")
    batch_per_w = idx_v.shape[0]
    seq = idx_v.shape[1]
    grows = NB * seq
    groups = batch_per_w // NB
    pairs = groups // 2
    b_base = wid * batch_per_w
    row_base = b_base * seq

    pltpu.sync_copy(idx_hbm.at[pl.ds(b_base, batch_per_w)], idx_v)

    def issue_gathers(g, buf, sem):
        for r in range(NB):
            for off, ln in SPLITS:
                pltpu.async_copy(
                    table_hbm.at[idx_v.at[g * NB + r, pl.ds(off, ln)]],
                    rows_v.at[buf, pl.ds(r * seq + off, ln)],
                    sem)

    def drain(sem, buf):
        pltpu.make_async_copy(out_hbm.at[pl.ds(0, grows), pl.ds(0, D_EMB)],
                              rows_v.at[buf], sem).wait()

    def writeback(g, buf, sem):
        pltpu.async_copy(rows_v.at[buf],
                         out_hbm.at[pl.ds(row_base + g * grows, grows),
                                    pl.ds(0, D_EMB)], sem)

    issue_gathers(0, 0, gsem0)

    def body(i, carry):
        a = 2 * i
        drain(gsem0, 0)
        writeback(a, 0, wsem0)
        pl.when(i > 0)(lambda: drain(wsem1, 1))
        issue_gathers(a + 1, 1, gsem1)

        def advance():
            drain(wsem0, 0)
            issue_gathers(a + 2, 0, gsem0)
        pl.when(i < pairs - 1)(advance)

        drain(gsem1, 1)
        writeback(a + 1, 1, wsem1)
        return carry

    lax.fori_loop(0, pairs, body, 0)
    drain(wsem0, 0)
    drain(wsem1, 1)


@jax.jit
def kernel(input_ids, table):
    batch, seq = input_ids.shape
    assert batch % NW == 0 and (batch // NW) % (2 * NB) == 0
    batch_per_w = batch // NW

    mesh = plsc.VectorSubcoreMesh(core_axis_name="c", subcore_axis_name="s",
                                  num_cores=NC, num_subcores=NS)

    run_t = pl.kernel(
        _transpose_body,
        out_type=jax.ShapeDtypeStruct((VOCAB * D_EMB,), jnp.float32),
        mesh=mesh,
        scratch_types=[
            pltpu.VMEM((N_TILES, 8, BLK), jnp.float32),
            pltpu.VMEM((CHUNK * BLK * D_EMB,), jnp.float32),
            pltpu.VMEM((8, 8, TAIL), jnp.float32),
            pltpu.VMEM((TAIL * D_EMB,), jnp.float32),
            pltpu.SemaphoreType.DMA,
        ],
        compiler_params=pltpu.CompilerParams(use_tc_tiling_on_sc=True,
                                             needs_layout_passes=False,
                                             disable_bounds_checks=True),
    )
    table_lin = run_t(table.T).reshape(VOCAB, D_EMB)

    run = pl.kernel(
        _gather_body,
        out_type=jax.ShapeDtypeStruct((batch * seq, 2 * D_EMB), jnp.float32),
        mesh=mesh,
        scratch_types=[
            pltpu.VMEM((batch_per_w, seq), jnp.int32),
            pltpu.VMEM((2, NB * seq, D_EMB), jnp.float32),
            pltpu.SemaphoreType.DMA,
            pltpu.SemaphoreType.DMA,
            pltpu.SemaphoreType.DMA,
            pltpu.SemaphoreType.DMA,
        ],
        compiler_params=pltpu.CompilerParams(use_tc_tiling_on_sc=False),
    )
    out = run(input_ids.astype(jnp.int32), table_lin)
    return out[:, :D_EMB].reshape(batch, seq, D_EMB)


# speedup vs baseline: 1.1537x; 1.1536x over previous
import jax
import jax.numpy as jnp
from jax import lax
from jax.experimental import pallas as pl
from jax.experimental.pallas import tpu as pltpu
from jax.experimental.pallas import tpu_sc as plsc

D_EMB = 64
NC, NS = 2, 16
NW = NC * NS
NB = 2
SPLITS = ((0, 128), (128, 72))


def _gather_body(idx_hbm, table_hbm, out_hbm, idx_v, rows_v,
                 gsem0, gsem1, wsem0, wsem1):
    wid = lax.axis_index("s") * NC + lax.axis_index("c")
    batch_per_w = idx_v.shape[0]
    seq = idx_v.shape[1]
    grows = NB * seq
    groups = batch_per_w // NB
    pairs = groups // 2
    b_base = wid * batch_per_w
    row_base = b_base * seq

    pltpu.sync_copy(idx_hbm.at[pl.ds(b_base, batch_per_w)], idx_v)

    def issue_gathers(g, buf, sem):
        for r in range(NB):
            for off, ln in SPLITS:
                pltpu.async_copy(
                    table_hbm.at[idx_v.at[g * NB + r, pl.ds(off, ln)]],
                    rows_v.at[buf, pl.ds(r * seq + off, ln)],
                    sem)

    def drain(sem, buf):
        pltpu.make_async_copy(out_hbm.at[pl.ds(0, grows), pl.ds(0, D_EMB)],
                              rows_v.at[buf], sem).wait()

    def writeback(g, buf, sem):
        pltpu.async_copy(rows_v.at[buf],
                         out_hbm.at[pl.ds(row_base + g * grows, grows),
                                    pl.ds(0, D_EMB)], sem)

    issue_gathers(0, 0, gsem0)

    def body(i, carry):
        a = 2 * i
        drain(gsem0, 0)
        writeback(a, 0, wsem0)
        pl.when(i > 0)(lambda: drain(wsem1, 1))
        issue_gathers(a + 1, 1, gsem1)

        def advance():
            drain(wsem0, 0)
            issue_gathers(a + 2, 0, gsem0)
        pl.when(i < pairs - 1)(advance)

        drain(gsem1, 1)
        writeback(a + 1, 1, wsem1)
        return carry

    lax.fori_loop(0, pairs, body, 0)
    drain(wsem0, 0)
    drain(wsem1, 1)


@jax.jit
def kernel(input_ids, table):
    batch, seq = input_ids.shape
    assert batch % NW == 0 and (batch // NW) % (2 * NB) == 0
    batch_per_w = batch // NW

    mesh = plsc.VectorSubcoreMesh(core_axis_name="c", subcore_axis_name="s",
                                  num_cores=NC, num_subcores=NS)
    run = pl.kernel(
        _gather_body,
        out_type=jax.ShapeDtypeStruct((batch * seq, 2 * D_EMB), jnp.float32),
        mesh=mesh,
        scratch_types=[
            pltpu.VMEM((batch_per_w, seq), jnp.int32),
            pltpu.VMEM((2, NB * seq, D_EMB), jnp.float32),
            pltpu.SemaphoreType.DMA,
            pltpu.SemaphoreType.DMA,
            pltpu.SemaphoreType.DMA,
            pltpu.SemaphoreType.DMA,
        ],
        compiler_params=pltpu.CompilerParams(use_tc_tiling_on_sc=False),
    )
    out = run(input_ids.astype(jnp.int32), table)
    return out[:, :D_EMB].reshape(batch, seq, D_EMB)
